# baseline (speedup 1.0000x reference)
import functools

import jax
import jax.numpy as jnp
from jax import lax
from jax.experimental import pallas as pl
from jax.experimental.pallas import tpu as pltpu
from jax.experimental.pallas import tpu_sc as plsc


@functools.partial(jax.jit, static_argnums=(1, 2, 3, 4, 5))
def _encode(args, E, D, s1, s2, NV):
    ea0, ea1, ea2, T = args
    info = plsc.get_sparse_core_info()
    NC, NS, L = info.num_cores, info.num_subcores, info.num_lanes
    NW = NC * NS
    chunk = E // NW
    SB = 80
    NB = chunk // SB
    NP = NB // 2
    NSLOT = 2
    G = SB // L

    mesh = plsc.VectorSubcoreMesh(core_axis_name="c", subcore_axis_name="s")

    @functools.partial(
        pl.kernel,
        mesh=mesh,
        out_type=jax.ShapeDtypeStruct((E, D), jnp.float32),
        scratch_types=[
            pltpu.VMEM((chunk,), jnp.int32),
            pltpu.VMEM((chunk,), jnp.int32),
            pltpu.VMEM((chunk,), jnp.int32),
            pltpu.VMEM((NB, SB), jnp.int32),
            pltpu.VMEM((NSLOT, 2 * SB, D), jnp.float32),
            pltpu.VMEM_SHARED((NS * (L // 4) * NV, D), jnp.float32),
            pltpu.SemaphoreType.DMA,
            pltpu.SemaphoreType.DMA,
        ],
    )
    def run(e0_hbm, e1_hbm, e2_hbm, t_hbm, out_hbm,
            e0_v, e1_v, e2_v, idx_v, rows_v, t_sp, sem_g, sem_w):
        sid = lax.axis_index("s")
        wid = sid * NC + lax.axis_index("c")
        base = wid * chunk
        toff = sid * (L // 4) * NV
        pltpu.sync_copy(e0_hbm.at[pl.ds(base, chunk)], e0_v)
        pltpu.sync_copy(e1_hbm.at[pl.ds(base, chunk)], e1_v)
        pltpu.sync_copy(e2_hbm.at[pl.ds(base, chunk)], e2_v)
        pltpu.sync_copy(t_hbm.at[pl.ds(toff, (L // 4) * NV)],
                        t_sp.at[pl.ds(toff, (L // 4) * NV)])

        lane_off = toff + lax.shift_right_logical(
            lax.iota(jnp.int32, L), 2) * NV

        def iblock(t, c):
            def group(g, c2):
                p = t * SB + g * L
                a0 = e0_v[pl.ds(p, L)]
                a1 = e1_v[pl.ds(p, L)]
                a2 = e2_v[pl.ds(p, L)]
                idx_v[t, pl.ds(g * L, L)] = a0 * s1 + a1 * s2 + a2 + lane_off
                return c2
            return lax.fori_loop(0, G, group, c)

        lax.fori_loop(0, NB, iblock, 0)

        def gather(p):
            s = lax.rem(p, NSLOT)
            pltpu.async_copy(
                t_sp.at[idx_v.at[2 * p]], rows_v.at[s, pl.ds(0, SB)], sem_g)
            pltpu.async_copy(
                t_sp.at[idx_v.at[2 * p + 1]], rows_v.at[s, pl.ds(SB, SB)],
                sem_g)

        def gather_wait(p):
            s = lax.rem(p, NSLOT)
            pltpu.make_async_copy(
                t_sp.at[idx_v.at[2 * p]], rows_v.at[s, pl.ds(0, SB)],
                sem_g).wait()
            pltpu.make_async_copy(
                t_sp.at[idx_v.at[2 * p + 1]], rows_v.at[s, pl.ds(SB, SB)],
                sem_g).wait()

        def wout(p):
            pltpu.async_copy(
                rows_v.at[lax.rem(p, NSLOT)],
                out_hbm.at[pl.ds(base + p * 2 * SB, 2 * SB)], sem_w)

        def wout_wait(p):
            pltpu.make_async_copy(
                rows_v.at[lax.rem(p, NSLOT)],
                out_hbm.at[pl.ds(base + p * 2 * SB, 2 * SB)], sem_w).wait()

        for p in range(NSLOT):
            gather(p)

        def step(p, c):
            @pl.when(p >= 1)
            def _():
                wout_wait(p - 1)

            @pl.when(jnp.logical_and(p >= 1, p - 1 + NSLOT < NP))
            def _():
                gather(p - 1 + NSLOT)

            gather_wait(p)
            wout(p)
            return c

        lax.fori_loop(0, NP, step, 0)
        wout_wait(NP - 1)

        if NB % 2:
            pltpu.async_copy(
                t_sp.at[idx_v.at[NB - 1]], rows_v.at[0, pl.ds(0, SB)], sem_g)
            pltpu.make_async_copy(
                t_sp.at[idx_v.at[NB - 1]], rows_v.at[0, pl.ds(0, SB)],
                sem_g).wait()
            pltpu.async_copy(
                rows_v.at[0, pl.ds(0, SB)],
                out_hbm.at[pl.ds(base + (NB - 1) * SB, SB)], sem_w)
            pltpu.make_async_copy(
                rows_v.at[0, pl.ds(0, SB)],
                out_hbm.at[pl.ds(base + (NB - 1) * SB, SB)], sem_w).wait()

    return run(ea0, ea1, ea2, T)


def kernel(edge_attr, W0, W1, W2):
    E = edge_attr.shape[0]
    D = W0.shape[1]
    n0, n1, n2 = W0.shape[0], W1.shape[0], W2.shape[0]
    NV = n0 * n1 * n2
    T = (W0[:, None, None, :] + W1[None, :, None, :] + W2[None, None, :, :])
    T = jnp.tile(T.reshape(NV, D), (16 * 4, 1))
    ea = edge_attr.astype(jnp.int32).T
    return _encode((ea[0], ea[1], ea[2], T), E, D, n1 * n2, n2, NV)


# speedup vs baseline: 1.0259x; 1.0259x over previous
import functools

import jax
import jax.numpy as jnp
from jax import lax
from jax.experimental import pallas as pl
from jax.experimental.pallas import tpu as pltpu
from jax.experimental.pallas import tpu_sc as plsc


@functools.partial(jax.jit, static_argnums=(1, 2, 3, 4, 5))
def _encode(args, E, D, s1, s2, NV):
    ea0, ea1, ea2, T = args
    info = plsc.get_sparse_core_info()
    NC, NS, L = info.num_cores, info.num_subcores, info.num_lanes
    NW = NC * NS
    chunk = E // NW
    SB = 80
    NB = chunk // SB
    NSLOT = 5
    G = SB // L

    mesh = plsc.VectorSubcoreMesh(core_axis_name="c", subcore_axis_name="s")

    @functools.partial(
        pl.kernel,
        mesh=mesh,
        out_type=jax.ShapeDtypeStruct((E, D), jnp.float32),
        scratch_types=[
            pltpu.VMEM((chunk,), jnp.int32),
            pltpu.VMEM((chunk,), jnp.int32),
            pltpu.VMEM((chunk,), jnp.int32),
            pltpu.VMEM((NB, SB), jnp.int32),
            pltpu.VMEM((NSLOT, SB, D), jnp.float32),
            pltpu.VMEM_SHARED((NS * (L // 4) * NV, D), jnp.float32),
            pltpu.SemaphoreType.DMA,
            pltpu.SemaphoreType.DMA,
        ],
    )
    def run(e0_hbm, e1_hbm, e2_hbm, t_hbm, out_hbm,
            e0_v, e1_v, e2_v, idx_v, rows_v, t_sp, sem_g, sem_w):
        sid = lax.axis_index("s")
        wid = sid * NC + lax.axis_index("c")
        base = wid * chunk
        toff = sid * (L // 4) * NV
        pltpu.sync_copy(e0_hbm.at[pl.ds(base, chunk)], e0_v)
        pltpu.sync_copy(e1_hbm.at[pl.ds(base, chunk)], e1_v)
        pltpu.sync_copy(e2_hbm.at[pl.ds(base, chunk)], e2_v)
        pltpu.sync_copy(t_hbm.at[pl.ds(toff, (L // 4) * NV)],
                        t_sp.at[pl.ds(toff, (L // 4) * NV)])

        lane_off = toff + lax.shift_right_logical(
            lax.iota(jnp.int32, L), 2) * NV

        def iblock(t, c):
            def group(g, c2):
                p = t * SB + g * L
                a0 = e0_v[pl.ds(p, L)]
                a1 = e1_v[pl.ds(p, L)]
                a2 = e2_v[pl.ds(p, L)]
                idx_v[t, pl.ds(g * L, L)] = a0 * s1 + a1 * s2 + a2 + lane_off
                return c2
            return lax.fori_loop(0, G, group, c)

        lax.fori_loop(0, NB, iblock, 0)

        def gather(t):
            pltpu.async_copy(
                t_sp.at[idx_v.at[t]], rows_v.at[lax.rem(t, NSLOT)], sem_g)

        def gather_wait(t):
            pltpu.make_async_copy(
                t_sp.at[idx_v.at[t]], rows_v.at[lax.rem(t, NSLOT)],
                sem_g).wait()

        def wout(t):
            pltpu.async_copy(
                rows_v.at[lax.rem(t, NSLOT)],
                out_hbm.at[pl.ds(base + t * SB, SB)], sem_w)

        def wout_wait(t):
            pltpu.make_async_copy(
                rows_v.at[lax.rem(t, NSLOT)],
                out_hbm.at[pl.ds(base + t * SB, SB)], sem_w).wait()

        for t in range(NSLOT):
            gather(t)

        def step(t, c):
            @pl.when(t >= 1)
            def _():
                wout_wait(t - 1)

            @pl.when(jnp.logical_and(t >= 1, t - 1 + NSLOT < NB))
            def _():
                gather(t - 1 + NSLOT)

            gather_wait(t)
            wout(t)
            return c

        lax.fori_loop(0, NB, step, 0)
        wout_wait(NB - 1)

    return run(ea0, ea1, ea2, T)


def kernel(edge_attr, W0, W1, W2):
    E = edge_attr.shape[0]
    D = W0.shape[1]
    n0, n1, n2 = W0.shape[0], W1.shape[0], W2.shape[0]
    NV = n0 * n1 * n2
    T = (W0[:, None, None, :] + W1[None, :, None, :] + W2[None, None, :, :])
    T = jnp.tile(T.reshape(NV, D), (16 * 4, 1))
    ea = edge_attr.astype(jnp.int32).T
    return _encode((ea[0], ea[1], ea[2], T), E, D, n1 * n2, n2, NV)
